# baseline (speedup 1.0000x reference)
import functools

import jax
import jax.numpy as jnp
from jax import lax
from jax.experimental import pallas as pl
from jax.experimental.pallas import tpu as pltpu
from jax.experimental.pallas import tpu_sc as plsc

_N = 10000
_E = 320000
_D = 128
_EPS = 1e-5

_NC = 2
_NS = 16
_NW = _NC * _NS
_C = 100
_KC = _E // (_NW * _C)
_RPT = _N // _NS

_mesh = plsc.VectorSubcoreMesh(core_axis_name="c", subcore_axis_name="s")


def _deg_body(dst_hbm, ones_hbm, degp_hbm, dst_v, ones_v, zbuf, deg_sh, sem):
    c = lax.axis_index("c")
    s = lax.axis_index("s")
    wid = s * _NC + c
    pltpu.sync_copy(dst_hbm.at[wid], dst_v)
    pltpu.sync_copy(ones_hbm, ones_v)

    @pl.loop(0, _RPT // 5)
    def _zrow(r):
        zbuf[r, pl.ds(0, 16)] = jnp.zeros((16,), jnp.float32)

    for t in range(5):
        pltpu.sync_copy(zbuf,
                        deg_sh.at[pl.ds(s * _RPT + t * (_RPT // 5), _RPT // 5)])
    plsc.subcore_barrier()

    @pl.loop(0, _KC)
    def _chunk(j):
        pltpu.async_copy(ones_v, deg_sh.at[dst_v.at[j]], sem, add=True)

    @pl.loop(0, _KC)
    def _drain(j):
        pltpu.make_async_copy(ones_v, deg_sh.at[dst_v.at[0]], sem).wait()

    plsc.subcore_barrier()
    pltpu.sync_copy(deg_sh.at[pl.ds(s * _RPT, _RPT)],
                    degp_hbm.at[c, pl.ds(s * _RPT, _RPT)])


def _agg_body(src_hbm, dst_hbm, y_hbm, accp_hbm,
              src_v, dst_v, rows0, rows1, acc_sh, gsem0, gsem1):
    c = lax.axis_index("c")
    s = lax.axis_index("s")
    wid = s * _NC + c
    pltpu.sync_copy(src_hbm.at[wid], src_v)
    pltpu.sync_copy(dst_hbm.at[wid], dst_v)

    @pl.loop(0, _C)
    def _zrow(r):
        for k in range(_D // 16):
            rows0[r, pl.ds(k * 16, 16)] = jnp.zeros((16,), jnp.float32)

    _nfull = _RPT // _C
    _rem = _RPT - _nfull * _C
    for t in range(_nfull):
        pltpu.sync_copy(rows0, acc_sh.at[pl.ds(s * _RPT + t * _C, _C)])
    if _rem:
        pltpu.sync_copy(rows0.at[pl.ds(0, _rem)],
                        acc_sh.at[pl.ds(s * _RPT + _nfull * _C, _rem)])

    pltpu.async_copy(y_hbm.at[src_v.at[0]], rows0, gsem0)
    pltpu.async_copy(y_hbm.at[src_v.at[1]], rows1, gsem1)
    plsc.subcore_barrier()

    @pl.loop(0, _KC, step=2)
    def _chunk(j):
        pltpu.make_async_copy(y_hbm.at[src_v.at[j]], rows0, gsem0).wait()
        pltpu.sync_copy(rows0, acc_sh.at[dst_v.at[j]], add=True)

        @pl.when(j + 2 < _KC)
        def _next0():
            pltpu.async_copy(y_hbm.at[src_v.at[j + 2]], rows0, gsem0)

        pltpu.make_async_copy(y_hbm.at[src_v.at[j + 1]], rows1, gsem1).wait()
        pltpu.sync_copy(rows1, acc_sh.at[dst_v.at[j + 1]], add=True)

        @pl.when(j + 2 < _KC)
        def _next1():
            pltpu.async_copy(y_hbm.at[src_v.at[j + 3]], rows1, gsem1)

    plsc.subcore_barrier()
    pltpu.sync_copy(acc_sh.at[pl.ds(s * _RPT, _RPT)],
                    accp_hbm.at[c, pl.ds(s * _RPT, _RPT)])


def _sc_degree(dst_w):
    ones = jnp.ones((_C, 16), jnp.float32)
    return pl.kernel(
        _deg_body,
        out_type=jax.ShapeDtypeStruct((_NC, _N, 16), jnp.float32),
        mesh=_mesh,
        scratch_types=[
            pltpu.VMEM((_KC, _C), jnp.int32),
            pltpu.VMEM((_C, 16), jnp.float32),
            pltpu.VMEM((_RPT // 5, 16), jnp.float32),
            pltpu.VMEM_SHARED((_N, 16), jnp.float32),
            pltpu.SemaphoreType.DMA,
        ],
        compiler_params=pltpu.CompilerParams(use_tc_tiling_on_sc=False),
    )(dst_w, ones)


def _sc_aggregate(src_w, dst_w, y):
    return pl.kernel(
        _agg_body,
        out_type=jax.ShapeDtypeStruct((_NC, _N, _D), jnp.float32),
        mesh=_mesh,
        scratch_types=[
            pltpu.VMEM((_KC, _C), jnp.int32),
            pltpu.VMEM((_KC, _C), jnp.int32),
            pltpu.VMEM((_C, _D), jnp.float32),
            pltpu.VMEM((_C, _D), jnp.float32),
            pltpu.VMEM_SHARED((_N, _D), jnp.float32),
            pltpu.SemaphoreType.DMA,
            pltpu.SemaphoreType.DMA,
        ],
        compiler_params=pltpu.CompilerParams(use_tc_tiling_on_sc=False),
    )(src_w, dst_w, y)


_R = 10000
_G = _N // _R


def _prep_body(degp_ref, x_ref, y_ref, dinv_ref):
    deg = 1.0 + degp_ref[0, :, 0:1] + degp_ref[1, :, 0:1]
    dinv = 1.0 / jnp.sqrt(deg)
    y_ref[...] = x_ref[...] * dinv
    dinv_ref[...] = jnp.broadcast_to(dinv, (_R, _D))


def _tc_prep(degp, x):
    return pl.pallas_call(
        _prep_body,
        grid=(_G,),
        in_specs=[
            pl.BlockSpec((_NC, _R, 16), lambda i: (0, i, 0)),
            pl.BlockSpec((_R, _D), lambda i: (i, 0)),
        ],
        out_specs=[
            pl.BlockSpec((_R, _D), lambda i: (i, 0)),
            pl.BlockSpec((_R, _D), lambda i: (i, 0)),
        ],
        out_shape=[
            jax.ShapeDtypeStruct((_N, _D), jnp.float32),
            jax.ShapeDtypeStruct((_N, _D), jnp.float32),
        ],
    )(degp, x)


def _mm_body(p_ref, y_ref, dinv_ref, w_ref, b_ref, h_ref, stats_ref):
    agg = dinv_ref[...] * (p_ref[0] + p_ref[1] + y_ref[...])
    h = jnp.dot(agg, w_ref[...], preferred_element_type=jnp.float32) + b_ref[...]
    h_ref[...] = h
    s1 = jnp.broadcast_to(jnp.sum(h, axis=0, keepdims=True), (4, _D))
    s2 = jnp.broadcast_to(jnp.sum(h * h, axis=0, keepdims=True), (4, _D))
    st = jnp.concatenate([s1, s2], axis=0)

    @pl.when(pl.program_id(0) == 0)
    def _init():
        stats_ref[...] = st

    @pl.when(pl.program_id(0) > 0)
    def _acc():
        stats_ref[...] += st


def _tc_matmul_stats(p, y, dinvb, W, b):
    return pl.pallas_call(
        _mm_body,
        grid=(_G,),
        in_specs=[
            pl.BlockSpec((_NC, _R, _D), lambda i: (0, i, 0)),
            pl.BlockSpec((_R, _D), lambda i: (i, 0)),
            pl.BlockSpec((_R, _D), lambda i: (i, 0)),
            pl.BlockSpec((_D, _D), lambda i: (0, 0)),
            pl.BlockSpec((1, _D), lambda i: (0, 0)),
        ],
        out_specs=[
            pl.BlockSpec((_R, _D), lambda i: (i, 0)),
            pl.BlockSpec((8, _D), lambda i: (0, 0)),
        ],
        out_shape=[
            jax.ShapeDtypeStruct((_N, _D), jnp.float32),
            jax.ShapeDtypeStruct((8, _D), jnp.float32),
        ],
    )(p, y, dinvb, W, b.reshape(1, _D))


def _bn_body(h_ref, stats_ref, gamma_ref, beta_ref, dinv_ref, out_ref, ynext_ref):
    inv_n = 1.0 / _N
    mean = stats_ref[0:1, :] * inv_n
    var = stats_ref[4:5, :] * inv_n - mean * mean
    scale = gamma_ref[...] / jnp.sqrt(var + _EPS)
    out = jnp.maximum((h_ref[...] - mean) * scale + beta_ref[...], 0.0)
    out_ref[...] = out
    if ynext_ref is not None:
        ynext_ref[...] = out * dinv_ref[...]


def _bn_body_last(h_ref, stats_ref, gamma_ref, beta_ref, dinv_ref, out_ref):
    _bn_body(h_ref, stats_ref, gamma_ref, beta_ref, dinv_ref, out_ref, None)


def _tc_bn_relu(h, stats, gamma, beta, dinvb, want_ynext):
    if want_ynext:
        body = _bn_body
        out_specs = [pl.BlockSpec((_R, _D), lambda i: (i, 0)),
                     pl.BlockSpec((_R, _D), lambda i: (i, 0))]
        out_shape = [jax.ShapeDtypeStruct((_N, _D), jnp.float32),
                     jax.ShapeDtypeStruct((_N, _D), jnp.float32)]
    else:
        body = _bn_body_last
        out_specs = [pl.BlockSpec((_R, _D), lambda i: (i, 0))]
        out_shape = [jax.ShapeDtypeStruct((_N, _D), jnp.float32)]
    return pl.pallas_call(
        body,
        grid=(_G,),
        in_specs=[
            pl.BlockSpec((_R, _D), lambda i: (i, 0)),
            pl.BlockSpec((8, _D), lambda i: (0, 0)),
            pl.BlockSpec((1, _D), lambda i: (0, 0)),
            pl.BlockSpec((1, _D), lambda i: (0, 0)),
            pl.BlockSpec((_R, _D), lambda i: (i, 0)),
        ],
        out_specs=out_specs,
        out_shape=out_shape,
    )(h, stats, gamma.reshape(1, _D), beta.reshape(1, _D), dinvb)


def kernel(node_emb, edge_index, W1, b1, gamma1, beta1, W2, b2, gamma2, beta2):
    src_w = edge_index[0].reshape(_NW, _KC, _C)
    dst_w = edge_index[1].reshape(_NW, _KC, _C)

    degp = _sc_degree(dst_w)
    y1, dinvb = _tc_prep(degp, node_emb)

    p1 = _sc_aggregate(src_w, dst_w, y1)
    h1, stats1 = _tc_matmul_stats(p1, y1, dinvb, W1, b1)
    _, y2 = _tc_bn_relu(h1, stats1, gamma1, beta1, dinvb, want_ynext=True)

    p2 = _sc_aggregate(src_w, dst_w, y2)
    h2, stats2 = _tc_matmul_stats(p2, y2, dinvb, W2, b2)
    (out,) = _tc_bn_relu(h2, stats2, gamma2, beta2, dinvb, want_ynext=False)
    return out


# speedup vs baseline: 1.0176x; 1.0176x over previous
import functools

import jax
import jax.numpy as jnp
from jax import lax
from jax.experimental import pallas as pl
from jax.experimental.pallas import tpu as pltpu
from jax.experimental.pallas import tpu_sc as plsc

_N = 10000
_E = 320000
_D = 128
_EPS = 1e-5

_NC = 2
_NS = 16
_NW = _NC * _NS
_C = 100
_KC = _E // (_NW * _C)
_RPT = _N // _NS

_mesh = plsc.VectorSubcoreMesh(core_axis_name="c", subcore_axis_name="s")


def _deg_body(dst_hbm, ones_hbm, degp_hbm, dst_v, ones_v, zbuf, deg_sh, sem):
    c = lax.axis_index("c")
    s = lax.axis_index("s")
    wid = s * _NC + c
    pltpu.sync_copy(dst_hbm.at[wid], dst_v)
    pltpu.sync_copy(ones_hbm, ones_v)

    @pl.loop(0, _RPT // 5)
    def _zrow(r):
        zbuf[r, pl.ds(0, 16)] = jnp.zeros((16,), jnp.float32)

    for t in range(5):
        pltpu.sync_copy(zbuf,
                        deg_sh.at[pl.ds(s * _RPT + t * (_RPT // 5), _RPT // 5)])
    plsc.subcore_barrier()

    @pl.loop(0, _KC)
    def _chunk(j):
        pltpu.async_copy(ones_v, deg_sh.at[dst_v.at[j]], sem, add=True)

    @pl.loop(0, _KC)
    def _drain(j):
        pltpu.make_async_copy(ones_v, deg_sh.at[dst_v.at[0]], sem).wait()

    plsc.subcore_barrier()
    pltpu.sync_copy(deg_sh.at[pl.ds(s * _RPT, _RPT)],
                    degp_hbm.at[c, pl.ds(s * _RPT, _RPT)])


def _agg_body(src_hbm, dst_hbm, y_hbm, accp_hbm,
              src_v, dst_v, rows0, rows1, acc_sh, gsem0, gsem1):
    c = lax.axis_index("c")
    s = lax.axis_index("s")
    wid = s * _NC + c
    pltpu.sync_copy(src_hbm.at[wid], src_v)
    pltpu.sync_copy(dst_hbm.at[wid], dst_v)

    @pl.loop(0, _C)
    def _zrow(r):
        for k in range(_D // 16):
            rows0[r, pl.ds(k * 16, 16)] = jnp.zeros((16,), jnp.float32)

    _nfull = _RPT // _C
    _rem = _RPT - _nfull * _C
    for t in range(_nfull):
        pltpu.sync_copy(rows0, acc_sh.at[pl.ds(s * _RPT + t * _C, _C)])
    if _rem:
        pltpu.sync_copy(rows0.at[pl.ds(0, _rem)],
                        acc_sh.at[pl.ds(s * _RPT + _nfull * _C, _rem)])

    pltpu.async_copy(y_hbm.at[src_v.at[0]], rows0, gsem0)
    pltpu.async_copy(y_hbm.at[src_v.at[1]], rows1, gsem1)
    plsc.subcore_barrier()

    @pl.loop(0, _KC, step=2)
    def _chunk(j):
        pltpu.make_async_copy(y_hbm.at[src_v.at[j]], rows0, gsem0).wait()
        pltpu.sync_copy(rows0, acc_sh.at[dst_v.at[j]], add=True)

        @pl.when(j + 2 < _KC)
        def _next0():
            pltpu.async_copy(y_hbm.at[src_v.at[j + 2]], rows0, gsem0)

        pltpu.make_async_copy(y_hbm.at[src_v.at[j + 1]], rows1, gsem1).wait()
        pltpu.sync_copy(rows1, acc_sh.at[dst_v.at[j + 1]], add=True)

        @pl.when(j + 2 < _KC)
        def _next1():
            pltpu.async_copy(y_hbm.at[src_v.at[j + 3]], rows1, gsem1)

    plsc.subcore_barrier()
    pltpu.sync_copy(acc_sh.at[pl.ds(s * _RPT, _RPT)],
                    accp_hbm.at[c, pl.ds(s * _RPT, _RPT)])


def _sc_degree(dst_w):
    ones = jnp.ones((_C, 16), jnp.float32)
    return pl.kernel(
        _deg_body,
        out_type=jax.ShapeDtypeStruct((_NC, _N, 16), jnp.float32),
        mesh=_mesh,
        scratch_types=[
            pltpu.VMEM((_KC, _C), jnp.int32),
            pltpu.VMEM((_C, 16), jnp.float32),
            pltpu.VMEM((_RPT // 5, 16), jnp.float32),
            pltpu.VMEM_SHARED((_N, 16), jnp.float32),
            pltpu.SemaphoreType.DMA,
        ],
        compiler_params=pltpu.CompilerParams(use_tc_tiling_on_sc=False),
    )(dst_w, ones)


def _sc_aggregate(src_w, dst_w, y):
    return pl.kernel(
        _agg_body,
        out_type=jax.ShapeDtypeStruct((_NC, _N, _D), jnp.float32),
        mesh=_mesh,
        scratch_types=[
            pltpu.VMEM((_KC, _C), jnp.int32),
            pltpu.VMEM((_KC, _C), jnp.int32),
            pltpu.VMEM((_C, _D), jnp.float32),
            pltpu.VMEM((_C, _D), jnp.float32),
            pltpu.VMEM_SHARED((_N, _D), jnp.float32),
            pltpu.SemaphoreType.DMA,
            pltpu.SemaphoreType.DMA,
        ],
        compiler_params=pltpu.CompilerParams(use_tc_tiling_on_sc=False),
    )(src_w, dst_w, y)


_R = 5000
_G = _N // _R


def _prep_body(degp_ref, x_ref, y_ref, dinv_ref):
    deg = 1.0 + degp_ref[0, :, 0:1] + degp_ref[1, :, 0:1]
    dinv = 1.0 / jnp.sqrt(deg)
    y_ref[...] = x_ref[...] * dinv
    dinv_ref[...] = jnp.broadcast_to(dinv, (_R, _D))


def _tc_prep(degp, x):
    return pl.pallas_call(
        _prep_body,
        grid=(_G,),
        in_specs=[
            pl.BlockSpec((_NC, _R, 16), lambda i: (0, i, 0)),
            pl.BlockSpec((_R, _D), lambda i: (i, 0)),
        ],
        out_specs=[
            pl.BlockSpec((_R, _D), lambda i: (i, 0)),
            pl.BlockSpec((_R, _D), lambda i: (i, 0)),
        ],
        out_shape=[
            jax.ShapeDtypeStruct((_N, _D), jnp.float32),
            jax.ShapeDtypeStruct((_N, _D), jnp.float32),
        ],
    )(degp, x)


def _mm_body(p_ref, y_ref, dinv_ref, w_ref, b_ref, h_ref, stats_ref):
    agg = dinv_ref[...] * (p_ref[0] + p_ref[1] + y_ref[...])
    h = jnp.dot(agg, w_ref[...], preferred_element_type=jnp.float32) + b_ref[...]
    h_ref[...] = h
    s1 = jnp.broadcast_to(jnp.sum(h, axis=0, keepdims=True), (4, _D))
    s2 = jnp.broadcast_to(jnp.sum(h * h, axis=0, keepdims=True), (4, _D))
    st = jnp.concatenate([s1, s2], axis=0)

    @pl.when(pl.program_id(0) == 0)
    def _init():
        stats_ref[...] = st

    @pl.when(pl.program_id(0) > 0)
    def _acc():
        stats_ref[...] += st


def _tc_matmul_stats(p, y, dinvb, W, b):
    return pl.pallas_call(
        _mm_body,
        grid=(_G,),
        in_specs=[
            pl.BlockSpec((_NC, _R, _D), lambda i: (0, i, 0)),
            pl.BlockSpec((_R, _D), lambda i: (i, 0)),
            pl.BlockSpec((_R, _D), lambda i: (i, 0)),
            pl.BlockSpec((_D, _D), lambda i: (0, 0)),
            pl.BlockSpec((1, _D), lambda i: (0, 0)),
        ],
        out_specs=[
            pl.BlockSpec((_R, _D), lambda i: (i, 0)),
            pl.BlockSpec((8, _D), lambda i: (0, 0)),
        ],
        out_shape=[
            jax.ShapeDtypeStruct((_N, _D), jnp.float32),
            jax.ShapeDtypeStruct((8, _D), jnp.float32),
        ],
    )(p, y, dinvb, W, b.reshape(1, _D))


def _bn_body(h_ref, stats_ref, gamma_ref, beta_ref, dinv_ref, out_ref, ynext_ref):
    inv_n = 1.0 / _N
    mean = stats_ref[0:1, :] * inv_n
    var = stats_ref[4:5, :] * inv_n - mean * mean
    scale = gamma_ref[...] / jnp.sqrt(var + _EPS)
    out = jnp.maximum((h_ref[...] - mean) * scale + beta_ref[...], 0.0)
    out_ref[...] = out
    if ynext_ref is not None:
        ynext_ref[...] = out * dinv_ref[...]


def _bn_body_last(h_ref, stats_ref, gamma_ref, beta_ref, dinv_ref, out_ref):
    _bn_body(h_ref, stats_ref, gamma_ref, beta_ref, dinv_ref, out_ref, None)


def _tc_bn_relu(h, stats, gamma, beta, dinvb, want_ynext):
    if want_ynext:
        body = _bn_body
        out_specs = [pl.BlockSpec((_R, _D), lambda i: (i, 0)),
                     pl.BlockSpec((_R, _D), lambda i: (i, 0))]
        out_shape = [jax.ShapeDtypeStruct((_N, _D), jnp.float32),
                     jax.ShapeDtypeStruct((_N, _D), jnp.float32)]
    else:
        body = _bn_body_last
        out_specs = [pl.BlockSpec((_R, _D), lambda i: (i, 0))]
        out_shape = [jax.ShapeDtypeStruct((_N, _D), jnp.float32)]
    return pl.pallas_call(
        body,
        grid=(_G,),
        in_specs=[
            pl.BlockSpec((_R, _D), lambda i: (i, 0)),
            pl.BlockSpec((8, _D), lambda i: (0, 0)),
            pl.BlockSpec((1, _D), lambda i: (0, 0)),
            pl.BlockSpec((1, _D), lambda i: (0, 0)),
            pl.BlockSpec((_R, _D), lambda i: (i, 0)),
        ],
        out_specs=out_specs,
        out_shape=out_shape,
    )(h, stats, gamma.reshape(1, _D), beta.reshape(1, _D), dinvb)


def kernel(node_emb, edge_index, W1, b1, gamma1, beta1, W2, b2, gamma2, beta2):
    src_w = edge_index[0].reshape(_NW, _KC, _C)
    dst_w = edge_index[1].reshape(_NW, _KC, _C)

    degp = _sc_degree(dst_w)
    y1, dinvb = _tc_prep(degp, node_emb)

    p1 = _sc_aggregate(src_w, dst_w, y1)
    h1, stats1 = _tc_matmul_stats(p1, y1, dinvb, W1, b1)
    _, y2 = _tc_bn_relu(h1, stats1, gamma1, beta1, dinvb, want_ynext=True)

    p2 = _sc_aggregate(src_w, dst_w, y2)
    h2, stats2 = _tc_matmul_stats(p2, y2, dinvb, W2, b2)
    (out,) = _tc_bn_relu(h2, stats2, gamma2, beta2, dinvb, want_ynext=False)
    return out
